# baseline (speedup 1.0000x reference)
import functools

import jax
import jax.numpy as jnp
from jax import lax
from jax.experimental import pallas as pl
from jax.experimental.pallas import tpu as pltpu
from jax.experimental.pallas import tpu_sc as plsc

NC = 2
NS = 16
NW = NC * NS

B1 = 16384
S = 20
D = 32
V = 1000000
BW = B1 // NW
CB = 256
NCH = BW // CB
NG = CB // 16

_mesh = plsc.VectorSubcoreMesh(core_axis_name="c", subcore_axis_name="s")


@functools.partial(
    pl.kernel,
    out_type=jax.ShapeDtypeStruct((S, D, B1), jnp.float32),
    mesh=_mesh,
    scratch_types=[
        pltpu.VMEM((S, BW), jnp.int32),
        pltpu.VMEM((2, 128), jnp.int32),
        pltpu.VMEM((2, 128), jnp.int32),
        pltpu.VMEM((CB, 128), jnp.float32),
        pltpu.VMEM((CB, 128), jnp.float32),
        pltpu.VMEM((D, CB), jnp.float32),
        pltpu.SemaphoreType.DMA,
        pltpu.SemaphoreType.DMA,
    ],
    compiler_params=pltpu.CompilerParams(needs_layout_passes=False),
)
def _gather_kernel(idx_hbm, tab_hbm, out_hbm, idx_v, i4a_v, i4b_v, g0, g1,
                   t_v, sem0, sem1):
    wid = lax.axis_index("s") * NC + lax.axis_index("c")
    b0 = pl.multiple_of(wid * BW, 128)

    pltpu.sync_copy(idx_hbm.at[:, pl.ds(b0, BW)], idx_v)

    def prep(s_, c, i4_v):
        for g in range(NG):
            vals = idx_v[s_, pl.ds(c * CB + g * 16, 16)]
            i4_v[g // 8, pl.ds((g % 8) * 16, 16)] = vals

    def fire(i4_v, buf, sem):
        for k in range(2):
            pltpu.async_copy(
                tab_hbm.at[i4_v.at[k]],
                buf.at[pl.ds(k * 128, 128)],
                sem,
            )

    def drain(buf, sem):
        pltpu.make_async_copy(tab_hbm.at[pl.ds(0, CB)], buf, sem).wait()

    def extract_write(s_, c, buf):
        for g in range(NG):
            row = jax.lax.iota(jnp.int32, 16) + (g * 16)
            for d in range(D):
                col = jnp.full((16,), d, jnp.int32)
                t_v[d, pl.ds(g * 16, 16)] = plsc.load_gather(buf, [row, col])
        off = pl.multiple_of(b0 + c * CB, 128)
        pltpu.sync_copy(t_v, out_hbm.at[s_, :, pl.ds(off, CB)])

    def seq_body(s_, carry):
        prep(s_, 0, i4a_v)
        fire(i4a_v, g0, sem0)
        prep(s_, 1, i4b_v)
        fire(i4b_v, g1, sem1)
        drain(g0, sem0)
        extract_write(s_, 0, g0)
        drain(g1, sem1)
        extract_write(s_, 1, g1)
        return carry

    lax.fori_loop(0, S, seq_body, 0)


def kernel(input_, weight):
    idx_t = input_.astype(jnp.int32).T
    tab = jnp.pad(weight, ((0, 0), (0, 128 - D)))
    out_t = _gather_kernel(idx_t, tab)
    return jnp.transpose(out_t, (2, 0, 1))


# speedup vs baseline: 1.1792x; 1.1792x over previous
import functools

import jax
import jax.numpy as jnp
from jax import lax
from jax.experimental import pallas as pl
from jax.experimental.pallas import tpu as pltpu
from jax.experimental.pallas import tpu_sc as plsc

NC = 2
NS = 16
NW = NC * NS

B1 = 16384
S = 20
D = 32
V = 1000000
BW = B1 // NW
CB = 256
NG = CB // 16

_mesh = plsc.VectorSubcoreMesh(core_axis_name="c", subcore_axis_name="s")


@functools.partial(
    pl.kernel,
    out_type=jax.ShapeDtypeStruct((S, D, B1), jnp.float32),
    mesh=_mesh,
    scratch_types=[
        pltpu.VMEM((S, BW), jnp.int32),
        pltpu.VMEM((2, 128), jnp.int32),
        pltpu.VMEM((2, 128), jnp.int32),
        pltpu.VMEM((CB, 128), jnp.float32),
        pltpu.VMEM((CB, 128), jnp.float32),
        pltpu.VMEM((D, CB), jnp.float32),
        pltpu.VMEM((D, CB), jnp.float32),
        pltpu.SemaphoreType.DMA,
        pltpu.SemaphoreType.DMA,
        pltpu.SemaphoreType.DMA,
        pltpu.SemaphoreType.DMA,
    ],
    compiler_params=pltpu.CompilerParams(needs_layout_passes=False),
)
def _gather_kernel(idx_hbm, tab_hbm, out_hbm, idx_v, i4a_v, i4b_v, g0, g1,
                   t0, t1, sem0, sem1, wsem0, wsem1):
    wid = lax.axis_index("s") * NC + lax.axis_index("c")
    b0 = pl.multiple_of(wid * BW, 128)

    pltpu.sync_copy(idx_hbm.at[:, pl.ds(b0, BW)], idx_v)

    def prep(s_, c, i4_v):
        @plsc.parallel_loop(0, NG, 1, unroll=4)
        def _(g):
            vals = idx_v[s_, pl.ds(c * CB + g * 16, 16)]
            i4_v[g // 8, pl.ds((g % 8) * 16, 16)] = vals

    def fire(i4_v, buf, sem):
        for k in range(2):
            pltpu.async_copy(
                tab_hbm.at[i4_v.at[k]],
                buf.at[pl.ds(k * 128, 128)],
                sem,
            )

    def drain(buf, sem):
        pltpu.make_async_copy(tab_hbm.at[pl.ds(0, CB)], buf, sem).wait()

    def extract(buf, t_v):
        @plsc.parallel_loop(0, NG, 1, unroll=4)
        def _(g):
            row = jax.lax.iota(jnp.int32, 16) + (g * 16)
            for d in range(D):
                col = jnp.full((16,), d, jnp.int32)
                t_v[d, pl.ds(g * 16, 16)] = plsc.load_gather(buf, [row, col])

    def wwait(t_v, wsem):
        pltpu.make_async_copy(t_v, out_hbm.at[0, :, pl.ds(0, CB)], wsem).wait()

    def seq_body(s_, carry):
        prep(s_, 0, i4a_v)
        fire(i4a_v, g0, sem0)
        prep(s_, 1, i4b_v)
        fire(i4b_v, g1, sem1)

        drain(g0, sem0)

        @pl.when(s_ > 0)
        def _():
            wwait(t0, wsem0)

        extract(g0, t0)
        off0 = pl.multiple_of(b0, 128)
        pltpu.async_copy(t0, out_hbm.at[s_, :, pl.ds(off0, CB)], wsem0)

        drain(g1, sem1)

        @pl.when(s_ > 0)
        def _():
            wwait(t1, wsem1)

        extract(g1, t1)
        off1 = pl.multiple_of(b0 + CB, 128)
        pltpu.async_copy(t1, out_hbm.at[s_, :, pl.ds(off1, CB)], wsem1)
        return carry

    lax.fori_loop(0, S, seq_body, 0)
    wwait(t0, wsem0)
    wwait(t1, wsem1)


def kernel(input_, weight):
    idx_t = input_.astype(jnp.int32).T
    tab = jnp.pad(weight, ((0, 0), (0, 128 - D)))
    out_t = _gather_kernel(idx_t, tab)
    return jnp.transpose(out_t, (2, 0, 1))


# speedup vs baseline: 1.4463x; 1.2266x over previous
import functools

import jax
import jax.numpy as jnp
from jax import lax
from jax.experimental import pallas as pl
from jax.experimental.pallas import tpu as pltpu
from jax.experimental.pallas import tpu_sc as plsc

NC = 2
NS = 16
NW = NC * NS

B1 = 16384
S = 20
D = 32
V = 1000000
V4 = V // 4
BW = B1 // NW
CB = 256
NG = CB // 16
NCHUNK = 2 * S

VBLK = 512
NBLK = 7812 * 128 // VBLK
KMAX = (NBLK + NW - 1) // NW * 2
VTAIL = 7812 * 128

_mesh = plsc.VectorSubcoreMesh(core_axis_name="c", subcore_axis_name="s")
_params = pltpu.CompilerParams(needs_layout_passes=False)


def _transpose_block(ib, ob, kmax):
    @plsc.parallel_loop(0, kmax, 1, unroll=4)
    def _(k):
        for gj in range(8):
            row = lax.iota(jnp.int32, 16) + (16 if gj % 2 else 0)
            col = lax.broadcast(4 * k + gj // 2, (16,))
            ob[k, pl.ds(gj * 16, 16)] = plsc.load_gather(ib, [row, col])


@functools.partial(
    pl.kernel,
    out_type=jax.ShapeDtypeStruct((V4, 128), jnp.float32),
    mesh=_mesh,
    scratch_types=[
        pltpu.VMEM((D, VBLK), jnp.float32),
        pltpu.VMEM((D, VBLK), jnp.float32),
        pltpu.VMEM((128, 128), jnp.float32),
        pltpu.VMEM((128, 128), jnp.float32),
        pltpu.SemaphoreType.DMA,
        pltpu.SemaphoreType.DMA,
        pltpu.SemaphoreType.DMA,
        pltpu.SemaphoreType.DMA,
    ],
    compiler_params=_params,
)
def _relayout_kernel(wt_hbm, tail_hbm, tab_hbm, in0, in1, ob0, ob1,
                     isem0, isem1, osem0, osem1):
    wid = lax.axis_index("s") * NC + lax.axis_index("c")

    def blk_of(k):
        return wid + k * NW

    def fire_in(k, ib, isem):
        @pl.when(blk_of(k) < NBLK)
        def _():
            off = pl.multiple_of(blk_of(k) * VBLK, 128)
            pltpu.async_copy(wt_hbm.at[:, pl.ds(off, VBLK)], ib, isem)

    def step(p, k, ib, ob, isem, osem):
        live = blk_of(k) < NBLK

        @pl.when(live)
        def _():
            pltpu.make_async_copy(
                wt_hbm.at[:, pl.ds(0, VBLK)], ib, isem).wait()

            @pl.when(p > 0)
            def _():
                pltpu.make_async_copy(
                    ob, tab_hbm.at[pl.ds(0, 128)], osem).wait()

            _transpose_block(ib, ob, 128)
            r0 = pl.multiple_of(blk_of(k) * 128, 8)
            pltpu.async_copy(ob, tab_hbm.at[pl.ds(r0, 128)], osem)

    fire_in(0, in0, isem0)
    fire_in(1, in1, isem1)

    def pair_body(p, carry):
        ka = 2 * p
        step(p, ka, in0, ob0, isem0, osem0)
        fire_in(ka + 2, in0, isem0)
        step(p, ka + 1, in1, ob1, isem1, osem1)
        fire_in(ka + 3, in1, isem1)
        return carry

    lax.fori_loop(0, KMAX // 2, pair_body, 0)
    pltpu.make_async_copy(ob0, tab_hbm.at[pl.ds(0, 128)], osem0).wait()
    pltpu.make_async_copy(ob1, tab_hbm.at[pl.ds(0, 128)], osem1).wait()

    @pl.when(wid == NW - 1)
    def _():
        pltpu.sync_copy(tail_hbm, tab_hbm.at[pl.ds(V4 - 16, 16)])


@functools.partial(
    pl.kernel,
    out_type=jax.ShapeDtypeStruct((S, D, B1), jnp.float32),
    mesh=_mesh,
    scratch_types=[
        pltpu.VMEM((S, BW), jnp.int32),
        pltpu.VMEM((2, 128), jnp.int32),
        pltpu.VMEM((2, 128), jnp.int32),
        pltpu.VMEM((CB, 128), jnp.float32),
        pltpu.VMEM((CB, 128), jnp.float32),
        pltpu.VMEM((D, CB), jnp.float32),
        pltpu.VMEM((D, CB), jnp.float32),
        pltpu.SemaphoreType.DMA,
        pltpu.SemaphoreType.DMA,
        pltpu.SemaphoreType.DMA,
        pltpu.SemaphoreType.DMA,
    ],
    compiler_params=_params,
)
def _gather_kernel(idx_hbm, tab_hbm, out_hbm, idx_v, i4a_v, i4b_v, g0, g1,
                   t0, t1, sem0, sem1, wsem0, wsem1):
    wid = lax.axis_index("s") * NC + lax.axis_index("c")
    b0 = pl.multiple_of(wid * BW, 128)

    pltpu.sync_copy(idx_hbm.at[:, pl.ds(b0, BW)], idx_v)

    def prep_fire(ck, i4_v, buf, sem):
        @pl.when(ck < NCHUNK)
        def _():
            s_ = ck // 2
            cc = ck % 2

            @plsc.parallel_loop(0, NG, 1, unroll=4)
            def _(g):
                vals = idx_v[s_, pl.ds(cc * CB + g * 16, 16)]
                i4_v[g // 8, pl.ds((g % 8) * 16, 16)] = (
                    lax.shift_right_logical(vals, 2))

            for k in range(2):
                pltpu.async_copy(
                    tab_hbm.at[i4_v.at[k]],
                    buf.at[pl.ds(k * 128, 128)],
                    sem,
                )

    def extract(s_, cc, buf, t_v):
        @plsc.parallel_loop(0, NG, 1, unroll=4)
        def _(g):
            vals = idx_v[s_, pl.ds(cc * CB + g * 16, 16)]
            qcol = (vals & 3) * 32
            row = jax.lax.iota(jnp.int32, 16) + (g * 16)
            for d in range(D):
                t_v[d, pl.ds(g * 16, 16)] = plsc.load_gather(
                    buf, [row, qcol + d])

    def wwait(t_v, wsem):
        pltpu.make_async_copy(t_v, out_hbm.at[0, :, pl.ds(0, CB)], wsem).wait()

    def step(p, ck, i4_v, buf, t_v, sem, wsem):
        pltpu.make_async_copy(tab_hbm.at[pl.ds(0, CB)], buf, sem).wait()

        @pl.when(p > 0)
        def _():
            wwait(t_v, wsem)

        cc = ck % 2
        extract(ck // 2, cc, buf, t_v)
        off = pl.multiple_of(b0 + cc * CB, 128)
        pltpu.async_copy(t_v, out_hbm.at[ck // 2, :, pl.ds(off, CB)], wsem)

    prep_fire(0, i4a_v, g0, sem0)
    prep_fire(1, i4b_v, g1, sem1)

    def pair_body(p, carry):
        ck = 2 * p
        step(p, ck, i4a_v, g0, t0, sem0, wsem0)
        prep_fire(ck + 2, i4a_v, g0, sem0)
        step(p, ck + 1, i4b_v, g1, t1, sem1, wsem1)
        prep_fire(ck + 3, i4b_v, g1, sem1)
        return carry

    lax.fori_loop(0, NCHUNK // 2, pair_body, 0)
    wwait(t0, wsem0)
    wwait(t1, wsem1)


def kernel(input_, weight):
    idx_t = input_.astype(jnp.int32).T
    wt = weight.T
    tail = jnp.reshape(weight[VTAIL:, :], (16, 128))
    tab = _relayout_kernel(wt, tail)
    out_t = _gather_kernel(idx_t, tab)
    return jnp.transpose(out_t, (2, 0, 1))


# speedup vs baseline: 1.4528x; 1.0045x over previous
import functools

import jax
import jax.numpy as jnp
from jax import lax
from jax.experimental import pallas as pl
from jax.experimental.pallas import tpu as pltpu
from jax.experimental.pallas import tpu_sc as plsc

NC = 2
NS = 16
NW = NC * NS

B1 = 16384
S = 20
D = 32
V = 1000000
V4 = V // 4
BW = B1 // NW
CB = 256
NG = CB // 16
NCHUNK = 2 * S

VBLK = 512
NBLK = 7812 * 128 // VBLK
KMAX = (NBLK + NW - 1) // NW * 2
VTAIL = 7812 * 128

_mesh = plsc.VectorSubcoreMesh(core_axis_name="c", subcore_axis_name="s")
_params = pltpu.CompilerParams(needs_layout_passes=False)


def _transpose_block(ib, ob, kmax):
    row_lo = lax.iota(jnp.int32, 16)
    row_hi = row_lo + 16

    @plsc.parallel_loop(0, kmax, 1, unroll=8)
    def _(k):
        kbase = lax.broadcast(4 * k, (16,))
        for q in range(4):
            col = kbase + q
            ob[k, pl.ds(q * 32, 16)] = plsc.load_gather(ib, [row_lo, col])
            ob[k, pl.ds(q * 32 + 16, 16)] = plsc.load_gather(ib, [row_hi, col])


@functools.partial(
    pl.kernel,
    out_type=jax.ShapeDtypeStruct((V4, 128), jnp.float32),
    mesh=_mesh,
    scratch_types=[
        pltpu.VMEM((D, VBLK), jnp.float32),
        pltpu.VMEM((D, VBLK), jnp.float32),
        pltpu.VMEM((128, 128), jnp.float32),
        pltpu.VMEM((128, 128), jnp.float32),
        pltpu.SemaphoreType.DMA,
        pltpu.SemaphoreType.DMA,
        pltpu.SemaphoreType.DMA,
        pltpu.SemaphoreType.DMA,
    ],
    compiler_params=_params,
)
def _relayout_kernel(wt_hbm, tail_hbm, tab_hbm, in0, in1, ob0, ob1,
                     isem0, isem1, osem0, osem1):
    wid = lax.axis_index("s") * NC + lax.axis_index("c")

    def blk_of(k):
        return wid + k * NW

    def fire_in(k, ib, isem):
        @pl.when(blk_of(k) < NBLK)
        def _():
            off = pl.multiple_of(blk_of(k) * VBLK, 128)
            pltpu.async_copy(wt_hbm.at[:, pl.ds(off, VBLK)], ib, isem)

    def step(p, k, ib, ob, isem, osem):
        live = blk_of(k) < NBLK

        @pl.when(live)
        def _():
            pltpu.make_async_copy(
                wt_hbm.at[:, pl.ds(0, VBLK)], ib, isem).wait()

            @pl.when(p > 0)
            def _():
                pltpu.make_async_copy(
                    ob, tab_hbm.at[pl.ds(0, 128)], osem).wait()

            _transpose_block(ib, ob, 128)
            r0 = pl.multiple_of(blk_of(k) * 128, 8)
            pltpu.async_copy(ob, tab_hbm.at[pl.ds(r0, 128)], osem)

    fire_in(0, in0, isem0)
    fire_in(1, in1, isem1)

    def pair_body(p, carry):
        ka = 2 * p
        step(p, ka, in0, ob0, isem0, osem0)
        fire_in(ka + 2, in0, isem0)
        step(p, ka + 1, in1, ob1, isem1, osem1)
        fire_in(ka + 3, in1, isem1)
        return carry

    lax.fori_loop(0, KMAX // 2, pair_body, 0)
    pltpu.make_async_copy(ob0, tab_hbm.at[pl.ds(0, 128)], osem0).wait()
    pltpu.make_async_copy(ob1, tab_hbm.at[pl.ds(0, 128)], osem1).wait()

    @pl.when(wid == NW - 1)
    def _():
        pltpu.sync_copy(tail_hbm, tab_hbm.at[pl.ds(V4 - 16, 16)])


@functools.partial(
    pl.kernel,
    out_type=jax.ShapeDtypeStruct((S, D, B1), jnp.float32),
    mesh=_mesh,
    scratch_types=[
        pltpu.VMEM((S, BW), jnp.int32),
        pltpu.VMEM((2, 128), jnp.int32),
        pltpu.VMEM((2, 128), jnp.int32),
        pltpu.VMEM((CB, 128), jnp.float32),
        pltpu.VMEM((CB, 128), jnp.float32),
        pltpu.VMEM((D, CB), jnp.float32),
        pltpu.VMEM((D, CB), jnp.float32),
        pltpu.SemaphoreType.DMA,
        pltpu.SemaphoreType.DMA,
        pltpu.SemaphoreType.DMA,
        pltpu.SemaphoreType.DMA,
    ],
    compiler_params=_params,
)
def _gather_kernel(idx_hbm, tab_hbm, out_hbm, idx_v, i4a_v, i4b_v, g0, g1,
                   t0, t1, sem0, sem1, wsem0, wsem1):
    wid = lax.axis_index("s") * NC + lax.axis_index("c")
    b0 = pl.multiple_of(wid * BW, 128)

    pltpu.sync_copy(idx_hbm.at[:, pl.ds(b0, BW)], idx_v)

    def prep_fire(ck, i4_v, buf, sem):
        @pl.when(ck < NCHUNK)
        def _():
            s_ = ck // 2
            cc = ck % 2

            @plsc.parallel_loop(0, NG, 1, unroll=4)
            def _(g):
                vals = idx_v[s_, pl.ds(cc * CB + g * 16, 16)]
                i4_v[g // 8, pl.ds((g % 8) * 16, 16)] = (
                    lax.shift_right_logical(vals, 2))

            for k in range(2):
                pltpu.async_copy(
                    tab_hbm.at[i4_v.at[k]],
                    buf.at[pl.ds(k * 128, 128)],
                    sem,
                )

    def extract(s_, cc, buf, t_v):
        @plsc.parallel_loop(0, NG, 1, unroll=4)
        def _(g):
            vals = idx_v[s_, pl.ds(cc * CB + g * 16, 16)]
            qcol = (vals & 3) * 32
            row = jax.lax.iota(jnp.int32, 16) + (g * 16)
            for d in range(D):
                t_v[d, pl.ds(g * 16, 16)] = plsc.load_gather(
                    buf, [row, qcol + d])

    def wwait(t_v, wsem):
        pltpu.make_async_copy(t_v, out_hbm.at[0, :, pl.ds(0, CB)], wsem).wait()

    def step(p, ck, i4_v, buf, t_v, sem, wsem):
        pltpu.make_async_copy(tab_hbm.at[pl.ds(0, CB)], buf, sem).wait()

        @pl.when(p > 0)
        def _():
            wwait(t_v, wsem)

        cc = ck % 2
        extract(ck // 2, cc, buf, t_v)
        off = pl.multiple_of(b0 + cc * CB, 128)
        pltpu.async_copy(t_v, out_hbm.at[ck // 2, :, pl.ds(off, CB)], wsem)

    prep_fire(0, i4a_v, g0, sem0)
    prep_fire(1, i4b_v, g1, sem1)

    def pair_body(p, carry):
        ck = 2 * p
        step(p, ck, i4a_v, g0, t0, sem0, wsem0)
        prep_fire(ck + 2, i4a_v, g0, sem0)
        step(p, ck + 1, i4b_v, g1, t1, sem1, wsem1)
        prep_fire(ck + 3, i4b_v, g1, sem1)
        return carry

    lax.fori_loop(0, NCHUNK // 2, pair_body, 0)
    wwait(t0, wsem0)
    wwait(t1, wsem1)


def kernel(input_, weight):
    idx_t = input_.astype(jnp.int32).T
    wt = weight.T
    tail = jnp.reshape(weight[VTAIL:, :], (16, 128))
    tab = _relayout_kernel(wt, tail)
    out_t = _gather_kernel(idx_t, tab)
    return jnp.transpose(out_t, (2, 0, 1))
